# baseline (speedup 1.0000x reference)
import functools

import jax
import jax.numpy as jnp
from jax import lax
from jax.experimental import pallas as pl
from jax.experimental.pallas import tpu as pltpu
from jax.experimental.pallas import tpu_sc as plsc

NC = 2
NS = 16
L = 16
K = 64
PG = 8
D = 128
DV = D // L


def _lanesum(v):
    for sh in (8, 4, 2, 1):
        idx = lax.rem(lax.iota(jnp.int32, L) + sh, jnp.full((L,), L, jnp.int32))
        v = v + jnp.take_along_axis(v, idx, axis=0)
    return v


def _ta(v, idx):
    return jnp.take_along_axis(v, idx, axis=0)


def _bc(v, i):
    return _ta(v, jnp.full((L,), i, jnp.int32))


def _compute_chunk(xj_b, base, xi_b, rs_v, A_v, C_v):
    lane = lax.iota(jnp.int32, L)
    rs_r = [rs_v[pl.ds(i * L, L)] for i in range(4)]
    A_r = [A_v[pl.ds(i * L, L)] for i in range(5)]
    C_r = [C_v[pl.ds(i * L, L)] for i in range(5)]
    s15, s31, s47 = _bc(rs_r[0], 15), _bc(rs_r[1], 15), _bc(rs_r[2], 15)
    for q in range(K // L):
        z = jnp.zeros((L,), jnp.float32)

        @plsc.parallel_loop(0, L, 1, unroll=4, carry=z)
        def _dot(l, se_acc):
            row = q * L + l
            rowj = base + row
            acc = xj_b[rowj, pl.ds(0, L)] * xi_b[row, pl.ds(0, L)]
            for dd in range(1, DV):
                acc = acc + (xj_b[rowj, pl.ds(dd * L, L)]
                             * xi_b[row, pl.ds(dd * L, L)])
            sp = _lanesum(acc)
            return jnp.where(lane == l, sp, se_acc)

        se = _dot * (1.0 / D)

        one = jnp.ones((L,), jnp.int32)
        zero = jnp.zeros((L,), jnp.int32)
        blk = (jnp.where(s15 < se, one, zero)
               + jnp.where(s31 < se, one, zero)
               + jnp.where(s47 < se, one, zero))

        def _probe(sel, refs, idx):
            v = jnp.where(sel >= 1, _ta(refs[1], idx), _ta(refs[0], idx))
            for i in range(2, len(refs)):
                v = jnp.where(sel >= i, _ta(refs[i], idx), v)
            return v

        kk = zero
        for b in (8, 4, 2, 1):
            v = _probe(blk, rs_r, kk + (b - 1))
            kk = jnp.where(v < se, kk + b, kk)
        kk = jnp.where(_probe(blk, rs_r, kk) < se, kk + 1, kk)
        k16 = blk * 16 + kk
        hi = jnp.right_shift(k16, 4)
        lo = jnp.bitwise_and(k16, 15)
        t = _probe(hi, A_r, lo) * se + _probe(hi, C_r, lo)
        wv = 1.0 + 1.0 / (1.0 + jnp.exp(-t))

        @plsc.parallel_loop(0, L, 1)
        def _scale(l):
            wl = jnp.take_along_axis(wv, jnp.broadcast_to(l, (L,)))
            row = base + q * L + l
            for dd in range(DV):
                xj_b[row, pl.ds(dd * L, L)] = xj_b[row, pl.ds(dd * L, L)] * wl


def _sc_kernel(acc_rows, groups_per_worker,
               x_hbm, src_hbm, dgat_hbm, dsca_hbm, rs_hbm, A_hbm, C_hbm,
               out_hbm,
               acc_sh, xj_ab, xi_a, xi_b, sidx_v, dgidx_v, dsidx_v,
               rs_v, A_v, C_v, saj, sai, sbj, sbi):
    c = lax.axis_index("c")
    s = lax.axis_index("s")
    wid = c * NS + s
    KK = 2 * K

    pltpu.sync_copy(rs_hbm, rs_v)
    pltpu.sync_copy(A_hbm, A_v)
    pltpu.sync_copy(C_hbm, C_v)

    zrow = jnp.zeros((L,), jnp.float32)

    def _zero_row(i, _):
        for dd in range(DV):
            xj_ab[i, pl.ds(dd * L, L)] = zrow
        return 0

    lax.fori_loop(0, KK, _zero_row, 0)
    rows_per_tile = acc_rows // NS
    zfull, zrem = rows_per_tile // KK, rows_per_tile % KK
    for z in range(zfull):
        pltpu.sync_copy(xj_ab, acc_sh.at[pl.ds(s * rows_per_tile + z * KK, KK)])
    if zrem:
        pltpu.sync_copy(
            xj_ab.at[pl.ds(0, zrem)],
            acc_sh.at[pl.ds(s * rows_per_tile + zfull * KK, zrem)])
    plsc.subcore_barrier()

    xj_lo = xj_ab.at[pl.ds(0, K)]
    xj_hi = xj_ab.at[pl.ds(K, K)]

    SUB = K // 4

    def _issue_xj(h, half, xj_dst, sj):
        for sub in range(4):
            pltpu.async_copy(
                x_hbm.at[sidx_v.at[h, pl.ds(half * K + sub * SUB, SUB)]],
                xj_dst.at[pl.ds(sub * SUB, SUB)], sj)

    def _issue_xi(h, half, xi_s, si):
        for sub in range(4):
            pltpu.async_copy(
                x_hbm.at[dgidx_v.at[h, pl.ds(half * K + sub * SUB, SUB)]],
                xi_s.at[pl.ds(sub * SUB, SUB)], si)

    def _issue_half(h, half, xj_dst, xi_s, sj, si):
        _issue_xj(h, half, xj_dst, sj)
        _issue_xi(h, half, xi_s, si)

    def _wait_half(h, half, xj_dst, xi_s, sj, si):
        for sub in range(4):
            pltpu.make_async_copy(
                x_hbm.at[sidx_v.at[h, pl.ds(half * K + sub * SUB, SUB)]],
                xj_dst.at[pl.ds(sub * SUB, SUB)], sj).wait()
            pltpu.make_async_copy(
                x_hbm.at[dgidx_v.at[h, pl.ds(half * K + sub * SUB, SUB)]],
                xi_s.at[pl.ds(sub * SUB, SUB)], si).wait()

    def _group(gg, _):
        base = wid * (groups_per_worker * PG) + gg * PG
        pltpu.sync_copy(src_hbm.at[pl.ds(base, PG)], sidx_v)
        pltpu.sync_copy(dgat_hbm.at[pl.ds(base, PG)], dgidx_v)
        pltpu.sync_copy(dsca_hbm.at[pl.ds(base, PG)], dsidx_v)
        _issue_half(0, 0, xj_lo, xi_a, saj, sai)

        def _pair(h, _):
            _issue_half(h, 1, xj_hi, xi_b, sbj, sbi)
            _wait_half(h, 0, xj_lo, xi_a, saj, sai)
            _compute_chunk(xj_ab, 0, xi_a, rs_v, A_v, C_v)

            @pl.when(h + 1 < PG)
            def _():
                _issue_xi(h + 1, 0, xi_a, sai)

            _wait_half(h, 1, xj_hi, xi_b, sbj, sbi)
            _compute_chunk(xj_ab, K, xi_b, rs_v, A_v, C_v)
            pltpu.sync_copy(xj_ab, acc_sh.at[dsidx_v.at[h]], add=True)

            @pl.when(h + 1 < PG)
            def _():
                _issue_xj(h + 1, 0, xj_lo, saj)

            return 0

        lax.fori_loop(0, PG, _pair, 0)
        return 0

    lax.fori_loop(0, groups_per_worker, _group, 0)
    plsc.subcore_barrier()

    pltpu.sync_copy(acc_sh.at[pl.ds(s * rows_per_tile, rows_per_tile)],
                    out_hbm.at[c, pl.ds(s * rows_per_tile, rows_per_tile)])


def _tc_combine(x_ref, p_ref, o_ref):
    o_ref[...] = x_ref[...] * (p_ref[0] + p_ref[1])


def _pwl_tables(W1, b1, W2, b2):
    w1v = W1.reshape(-1).astype(jnp.float32)
    w2v = W2.reshape(-1).astype(jnp.float32)
    b1v = b1.astype(jnp.float32)
    nz = w1v != 0.0
    const_t = jnp.sum(jnp.where(nz, 0.0, jnp.maximum(b1v, 0.0) * w2v))
    r = jnp.where(nz, -b1v / jnp.where(nz, w1v, 1.0), jnp.inf)
    sl = jnp.where(nz, w1v * w2v, 0.0)
    ic = jnp.where(nz, b1v * w2v, 0.0)
    order = jnp.argsort(r)
    rs = r[order]
    sl, ic = sl[order], ic[order]
    pos = jnp.take(w1v, order) > 0.0
    slp = jnp.where(pos, sl, 0.0)
    icp = jnp.where(pos, ic, 0.0)
    sln = jnp.where(pos, 0.0, sl)
    icn = jnp.where(pos, 0.0, ic)
    zero1 = jnp.zeros((1,), jnp.float32)
    Ap = jnp.concatenate([zero1, jnp.cumsum(slp)])
    Cp = jnp.concatenate([zero1, jnp.cumsum(icp)])
    An = jnp.concatenate([jnp.cumsum(sln[::-1])[::-1], zero1])
    Cn = jnp.concatenate([jnp.cumsum(icn[::-1])[::-1], zero1])
    A = Ap + An
    C = Cp + Cn + const_t + b2.reshape(()).astype(jnp.float32)
    H = w1v.shape[0]
    pad = -(-(H + 1) // L) * L - (H + 1)
    A = jnp.concatenate([A, jnp.zeros((pad,), jnp.float32)])
    C = jnp.concatenate([C, jnp.zeros((pad,), jnp.float32)])
    return rs, A, C


@jax.jit
def kernel(x, edge_index, W1, b1, W2, b2):
    n, d = x.shape
    e = edge_index.shape[1]
    assert d == D and n % NS == 0

    src = edge_index[0].astype(jnp.int32)
    dst = edge_index[1].astype(jnp.int32)

    n_workers = NC * NS
    KK = 2 * K
    ppw = -(-e // (n_workers * KK * PG)) * PG
    e_pad = n_workers * ppw * KK
    npad = e_pad - e
    src = jnp.concatenate([src, jnp.zeros((npad,), jnp.int32)])
    dgat = jnp.concatenate([dst, jnp.zeros((npad,), jnp.int32)])
    dsca = jnp.concatenate([dst, jnp.full((npad,), n, jnp.int32)])
    src2d = src.reshape(e_pad // KK, KK)
    dgat2d = dgat.reshape(e_pad // KK, KK)
    dsca2d = dsca.reshape(e_pad // KK, KK)

    acc_rows = -(-(n + 1) // (NS * 8)) * NS * 8

    rs, A, C = _pwl_tables(W1, b1, W2, b2)

    mesh = plsc.VectorSubcoreMesh(core_axis_name="c", subcore_axis_name="s")
    partials = pl.kernel(
        functools.partial(_sc_kernel, acc_rows, ppw // PG),
        out_type=jax.ShapeDtypeStruct((NC, acc_rows, D), jnp.float32),
        mesh=mesh,
        scratch_types=[
            pltpu.VMEM_SHARED((acc_rows, D), jnp.float32),
            pltpu.VMEM((KK, D), jnp.float32),
            pltpu.VMEM((K, D), jnp.float32),
            pltpu.VMEM((K, D), jnp.float32),
            pltpu.VMEM((PG, KK), jnp.int32),
            pltpu.VMEM((PG, KK), jnp.int32),
            pltpu.VMEM((PG, KK), jnp.int32),
            pltpu.VMEM(rs.shape, jnp.float32),
            pltpu.VMEM(A.shape, jnp.float32),
            pltpu.VMEM(C.shape, jnp.float32),
            pltpu.SemaphoreType.DMA,
            pltpu.SemaphoreType.DMA,
            pltpu.SemaphoreType.DMA,
            pltpu.SemaphoreType.DMA,
        ],
    )(x, src2d, dgat2d, dsca2d, rs, A, C)

    blk = 400
    out = pl.pallas_call(
        _tc_combine,
        grid=(n // blk,),
        in_specs=[
            pl.BlockSpec((blk, D), lambda i: (i, 0)),
            pl.BlockSpec((NC, blk, D), lambda i: (0, i, 0)),
        ],
        out_specs=pl.BlockSpec((blk, D), lambda i: (i, 0)),
        out_shape=jax.ShapeDtypeStruct((n, D), jnp.float32),
    )(x, partials)
    return out


# speedup vs baseline: 1.1181x; 1.1181x over previous
import functools

import jax
import jax.numpy as jnp
from jax import lax
from jax.experimental import pallas as pl
from jax.experimental.pallas import tpu as pltpu
from jax.experimental.pallas import tpu_sc as plsc

NC = 2
NS = 16
L = 16
K = 64
PG = 8
D = 128
DV = D // L


def _lanesum(v):
    for sh in (8, 4, 2, 1):
        idx = lax.rem(lax.iota(jnp.int32, L) + sh, jnp.full((L,), L, jnp.int32))
        v = v + jnp.take_along_axis(v, idx, axis=0)
    return v


def _ta(v, idx):
    return jnp.take_along_axis(v, idx, axis=0)


def _bc(v, i):
    return _ta(v, jnp.full((L,), i, jnp.int32))


def _compute_chunk(xj_b, base, xi_b, rs_v, A_v, C_v):
    lane = lax.iota(jnp.int32, L)
    rs_r = [rs_v[pl.ds(i * L, L)] for i in range(4)]
    A_r = [A_v[pl.ds(i * L, L)] for i in range(5)]
    C_r = [C_v[pl.ds(i * L, L)] for i in range(5)]
    s15, s31, s47 = _bc(rs_r[0], 15), _bc(rs_r[1], 15), _bc(rs_r[2], 15)
    for q in range(K // L):
        z = jnp.zeros((L,), jnp.float32)

        @plsc.parallel_loop(0, L, 1, unroll=4, carry=z)
        def _dot(l, se_acc):
            row = q * L + l
            rowj = base + row
            acc = xj_b[rowj, pl.ds(0, L)] * xi_b[row, pl.ds(0, L)]
            for dd in range(1, DV):
                acc = acc + (xj_b[rowj, pl.ds(dd * L, L)]
                             * xi_b[row, pl.ds(dd * L, L)])
            sp = _lanesum(acc)
            return jnp.where(lane == l, sp, se_acc)

        se = _dot * (1.0 / D)

        one = jnp.ones((L,), jnp.int32)
        zero = jnp.zeros((L,), jnp.int32)
        blk = (jnp.where(s15 < se, one, zero)
               + jnp.where(s31 < se, one, zero)
               + jnp.where(s47 < se, one, zero))

        def _probe(sel, refs, idx):
            v = jnp.where(sel >= 1, _ta(refs[1], idx), _ta(refs[0], idx))
            for i in range(2, len(refs)):
                v = jnp.where(sel >= i, _ta(refs[i], idx), v)
            return v

        kk = zero
        for b in (8, 4, 2, 1):
            v = _probe(blk, rs_r, kk + (b - 1))
            kk = jnp.where(v < se, kk + b, kk)
        kk = jnp.where(_probe(blk, rs_r, kk) < se, kk + 1, kk)
        k16 = blk * 16 + kk
        hi = jnp.right_shift(k16, 4)
        lo = jnp.bitwise_and(k16, 15)
        t = _probe(hi, A_r, lo) * se + _probe(hi, C_r, lo)
        wv = 1.0 + 1.0 / (1.0 + jnp.exp(-t))

        @plsc.parallel_loop(0, L, 1)
        def _scale(l):
            wl = jnp.take_along_axis(wv, jnp.broadcast_to(l, (L,)))
            row = base + q * L + l
            for dd in range(DV):
                xj_b[row, pl.ds(dd * L, L)] = xj_b[row, pl.ds(dd * L, L)] * wl


def _sc_kernel(acc_rows, ppw0, ppw1,
               x_hbm, src_hbm, dgat_hbm, dsca_hbm, rs_hbm, A_hbm, C_hbm,
               out_hbm,
               acc_sh, xj_ab, xi_a, xi_b, sidx_v, dgidx_v, dsidx_v,
               rs_v, A_v, C_v, saj, sai, sbj, sbi):
    c = lax.axis_index("c")
    s = lax.axis_index("s")
    KK = 2 * K
    worker_base = jnp.where(c == 0, s * ppw0, NS * ppw0 + s * ppw1)
    gpw = jnp.where(c == 0, ppw0 // PG, ppw1 // PG)

    pltpu.sync_copy(rs_hbm, rs_v)
    pltpu.sync_copy(A_hbm, A_v)
    pltpu.sync_copy(C_hbm, C_v)

    zrow = jnp.zeros((L,), jnp.float32)

    def _zero_row(i, _):
        for dd in range(DV):
            xj_ab[i, pl.ds(dd * L, L)] = zrow
        return 0

    lax.fori_loop(0, KK, _zero_row, 0)
    rows_per_tile = acc_rows // NS
    zfull, zrem = rows_per_tile // KK, rows_per_tile % KK
    for z in range(zfull):
        pltpu.sync_copy(xj_ab, acc_sh.at[pl.ds(s * rows_per_tile + z * KK, KK)])
    if zrem:
        pltpu.sync_copy(
            xj_ab.at[pl.ds(0, zrem)],
            acc_sh.at[pl.ds(s * rows_per_tile + zfull * KK, zrem)])
    plsc.subcore_barrier()

    xj_lo = xj_ab.at[pl.ds(0, K)]
    xj_hi = xj_ab.at[pl.ds(K, K)]

    SUB = K // 2

    def _issue_xj(h, half, xj_dst, sj):
        for sub in range(2):
            pltpu.async_copy(
                x_hbm.at[sidx_v.at[h, pl.ds(half * K + sub * SUB, SUB)]],
                xj_dst.at[pl.ds(sub * SUB, SUB)], sj)

    def _issue_xi(h, half, xi_s, si):
        for sub in range(2):
            pltpu.async_copy(
                x_hbm.at[dgidx_v.at[h, pl.ds(half * K + sub * SUB, SUB)]],
                xi_s.at[pl.ds(sub * SUB, SUB)], si)

    def _issue_half(h, half, xj_dst, xi_s, sj, si):
        _issue_xj(h, half, xj_dst, sj)
        _issue_xi(h, half, xi_s, si)

    def _wait_half(h, half, xj_dst, xi_s, sj, si):
        for sub in range(2):
            pltpu.make_async_copy(
                x_hbm.at[sidx_v.at[h, pl.ds(half * K + sub * SUB, SUB)]],
                xj_dst.at[pl.ds(sub * SUB, SUB)], sj).wait()
            pltpu.make_async_copy(
                x_hbm.at[dgidx_v.at[h, pl.ds(half * K + sub * SUB, SUB)]],
                xi_s.at[pl.ds(sub * SUB, SUB)], si).wait()

    def _group(gg, _):
        base = worker_base + gg * PG
        pltpu.sync_copy(src_hbm.at[pl.ds(base, PG)], sidx_v)
        pltpu.sync_copy(dgat_hbm.at[pl.ds(base, PG)], dgidx_v)
        pltpu.sync_copy(dsca_hbm.at[pl.ds(base, PG)], dsidx_v)
        _issue_half(0, 0, xj_lo, xi_a, saj, sai)

        def _pair(h, _):
            _issue_half(h, 1, xj_hi, xi_b, sbj, sbi)
            _wait_half(h, 0, xj_lo, xi_a, saj, sai)
            _compute_chunk(xj_ab, 0, xi_a, rs_v, A_v, C_v)

            @pl.when(h + 1 < PG)
            def _():
                _issue_xi(h + 1, 0, xi_a, sai)

            _wait_half(h, 1, xj_hi, xi_b, sbj, sbi)
            _compute_chunk(xj_ab, K, xi_b, rs_v, A_v, C_v)
            pltpu.sync_copy(xj_ab, acc_sh.at[dsidx_v.at[h]], add=True)

            @pl.when(h + 1 < PG)
            def _():
                _issue_xj(h + 1, 0, xj_lo, saj)

            return 0

        lax.fori_loop(0, PG, _pair, 0)
        return 0

    lax.fori_loop(0, gpw, _group, 0)
    plsc.subcore_barrier()

    pltpu.sync_copy(acc_sh.at[pl.ds(s * rows_per_tile, rows_per_tile)],
                    out_hbm.at[c, pl.ds(s * rows_per_tile, rows_per_tile)])


def _tc_combine(x_ref, p_ref, o_ref):
    o_ref[...] = x_ref[...] * (p_ref[0] + p_ref[1])


def _pwl_tables(W1, b1, W2, b2):
    w1v = W1.reshape(-1).astype(jnp.float32)
    w2v = W2.reshape(-1).astype(jnp.float32)
    b1v = b1.astype(jnp.float32)
    nz = w1v != 0.0
    const_t = jnp.sum(jnp.where(nz, 0.0, jnp.maximum(b1v, 0.0) * w2v))
    r = jnp.where(nz, -b1v / jnp.where(nz, w1v, 1.0), jnp.inf)
    sl = jnp.where(nz, w1v * w2v, 0.0)
    ic = jnp.where(nz, b1v * w2v, 0.0)
    order = jnp.argsort(r)
    rs = r[order]
    sl, ic = sl[order], ic[order]
    pos = jnp.take(w1v, order) > 0.0
    slp = jnp.where(pos, sl, 0.0)
    icp = jnp.where(pos, ic, 0.0)
    sln = jnp.where(pos, 0.0, sl)
    icn = jnp.where(pos, 0.0, ic)
    zero1 = jnp.zeros((1,), jnp.float32)
    Ap = jnp.concatenate([zero1, jnp.cumsum(slp)])
    Cp = jnp.concatenate([zero1, jnp.cumsum(icp)])
    An = jnp.concatenate([jnp.cumsum(sln[::-1])[::-1], zero1])
    Cn = jnp.concatenate([jnp.cumsum(icn[::-1])[::-1], zero1])
    A = Ap + An
    C = Cp + Cn + const_t + b2.reshape(()).astype(jnp.float32)
    H = w1v.shape[0]
    pad = -(-(H + 1) // L) * L - (H + 1)
    A = jnp.concatenate([A, jnp.zeros((pad,), jnp.float32)])
    C = jnp.concatenate([C, jnp.zeros((pad,), jnp.float32)])
    return rs, A, C


@jax.jit
def kernel(x, edge_index, W1, b1, W2, b2):
    n, d = x.shape
    e = edge_index.shape[1]
    assert d == D and n % NS == 0

    src = edge_index[0].astype(jnp.int32)
    dst = edge_index[1].astype(jnp.int32)

    KK = 2 * K
    T = -(-e // (NS * KK * PG)) * PG
    ppw0 = max(PG, (int(T * 0.65) // PG) * PG)
    ppw1 = T - ppw0
    assert ppw1 >= PG
    e_pad = NS * T * KK
    npad = e_pad - e
    src = jnp.concatenate([src, jnp.zeros((npad,), jnp.int32)])
    dgat = jnp.concatenate([dst, jnp.zeros((npad,), jnp.int32)])
    dsca = jnp.concatenate([dst, jnp.full((npad,), n, jnp.int32)])
    src2d = src.reshape(e_pad // KK, KK)
    dgat2d = dgat.reshape(e_pad // KK, KK)
    dsca2d = dsca.reshape(e_pad // KK, KK)

    acc_rows = -(-(n + 1) // (NS * 8)) * NS * 8

    rs, A, C = _pwl_tables(W1, b1, W2, b2)

    mesh = plsc.VectorSubcoreMesh(core_axis_name="c", subcore_axis_name="s")
    partials = pl.kernel(
        functools.partial(_sc_kernel, acc_rows, ppw0, ppw1),
        out_type=jax.ShapeDtypeStruct((NC, acc_rows, D), jnp.float32),
        mesh=mesh,
        scratch_types=[
            pltpu.VMEM_SHARED((acc_rows, D), jnp.float32),
            pltpu.VMEM((KK, D), jnp.float32),
            pltpu.VMEM((K, D), jnp.float32),
            pltpu.VMEM((K, D), jnp.float32),
            pltpu.VMEM((PG, KK), jnp.int32),
            pltpu.VMEM((PG, KK), jnp.int32),
            pltpu.VMEM((PG, KK), jnp.int32),
            pltpu.VMEM(rs.shape, jnp.float32),
            pltpu.VMEM(A.shape, jnp.float32),
            pltpu.VMEM(C.shape, jnp.float32),
            pltpu.SemaphoreType.DMA,
            pltpu.SemaphoreType.DMA,
            pltpu.SemaphoreType.DMA,
            pltpu.SemaphoreType.DMA,
        ],
    )(x, src2d, dgat2d, dsca2d, rs, A, C)

    blk = 400
    out = pl.pallas_call(
        _tc_combine,
        grid=(n // blk,),
        in_specs=[
            pl.BlockSpec((blk, D), lambda i: (i, 0)),
            pl.BlockSpec((NC, blk, D), lambda i: (0, i, 0)),
        ],
        out_specs=pl.BlockSpec((blk, D), lambda i: (i, 0)),
        out_shape=jax.ShapeDtypeStruct((n, D), jnp.float32),
    )(x, partials)
    return out


# speedup vs baseline: 1.1734x; 1.0495x over previous
import functools

import jax
import jax.numpy as jnp
from jax import lax
from jax.experimental import pallas as pl
from jax.experimental.pallas import tpu as pltpu
from jax.experimental.pallas import tpu_sc as plsc

NC = 2
NS = 16
L = 16
K = 64
PG = 8
D = 128
DV = D // L


def _lanesum(v):
    for sh in (8, 4, 2, 1):
        idx = lax.rem(lax.iota(jnp.int32, L) + sh, jnp.full((L,), L, jnp.int32))
        v = v + jnp.take_along_axis(v, idx, axis=0)
    return v


def _ta(v, idx):
    return jnp.take_along_axis(v, idx, axis=0)


def _bc(v, i):
    return _ta(v, jnp.full((L,), i, jnp.int32))


def _compute_chunk(xj_b, base, xi_b, rs_v, A_v, C_v):
    lane = lax.iota(jnp.int32, L)
    rs_r = [rs_v[pl.ds(i * L, L)] for i in range(4)]
    A_r = [A_v[pl.ds(i * L, L)] for i in range(5)]
    C_r = [C_v[pl.ds(i * L, L)] for i in range(5)]
    s15, s31, s47 = _bc(rs_r[0], 15), _bc(rs_r[1], 15), _bc(rs_r[2], 15)
    for q in range(K // L):
        z = jnp.zeros((L,), jnp.float32)

        @plsc.parallel_loop(0, L, 1, unroll=4, carry=z)
        def _dot(l, se_acc):
            row = q * L + l
            rowj = base + row
            acc = xj_b[rowj, pl.ds(0, L)] * xi_b[row, pl.ds(0, L)]
            for dd in range(1, DV):
                acc = acc + (xj_b[rowj, pl.ds(dd * L, L)]
                             * xi_b[row, pl.ds(dd * L, L)])
            sp = _lanesum(acc)
            return jnp.where(lane == l, sp, se_acc)

        se = _dot * (1.0 / D)

        one = jnp.ones((L,), jnp.int32)
        zero = jnp.zeros((L,), jnp.int32)
        blk = (jnp.where(s15 < se, one, zero)
               + jnp.where(s31 < se, one, zero)
               + jnp.where(s47 < se, one, zero))

        def _probe(sel, refs, idx):
            v = jnp.where(sel >= 1, _ta(refs[1], idx), _ta(refs[0], idx))
            for i in range(2, len(refs)):
                v = jnp.where(sel >= i, _ta(refs[i], idx), v)
            return v

        kk = zero
        for b in (8, 4, 2, 1):
            v = _probe(blk, rs_r, kk + (b - 1))
            kk = jnp.where(v < se, kk + b, kk)
        kk = jnp.where(_probe(blk, rs_r, kk) < se, kk + 1, kk)
        k16 = blk * 16 + kk
        hi = jnp.right_shift(k16, 4)
        lo = jnp.bitwise_and(k16, 15)
        t = _probe(hi, A_r, lo) * se + _probe(hi, C_r, lo)
        wv = 1.0 + 1.0 / (1.0 + jnp.exp(-t))

        @plsc.parallel_loop(0, L, 1)
        def _scale(l):
            wl = jnp.take_along_axis(wv, jnp.broadcast_to(l, (L,)))
            row = base + q * L + l
            for dd in range(DV):
                xj_b[row, pl.ds(dd * L, L)] = xj_b[row, pl.ds(dd * L, L)] * wl


def _sc_kernel(acc_rows, ppw0, ppw1,
               x_hbm, src_hbm, dgat_hbm, dsca_hbm, rs_hbm, A_hbm, C_hbm,
               out_hbm,
               acc_sh, xj_ab, xi_a, xi_b, sidx_v, dgidx_v, dsidx_v,
               rs_v, A_v, C_v, saj, sai, sbj, sbi):
    c = lax.axis_index("c")
    s = lax.axis_index("s")
    KK = 2 * K
    worker_base = jnp.where(c == 0, s * ppw0, NS * ppw0 + s * ppw1)
    gpw = jnp.where(c == 0, ppw0 // PG, ppw1 // PG)

    pltpu.sync_copy(rs_hbm, rs_v)
    pltpu.sync_copy(A_hbm, A_v)
    pltpu.sync_copy(C_hbm, C_v)

    zrow = jnp.zeros((L,), jnp.float32)

    def _zero_row(i, _):
        for dd in range(DV):
            xj_ab[i, pl.ds(dd * L, L)] = zrow
        return 0

    lax.fori_loop(0, KK, _zero_row, 0)
    rows_per_tile = acc_rows // NS
    zfull, zrem = rows_per_tile // KK, rows_per_tile % KK
    for z in range(zfull):
        pltpu.sync_copy(xj_ab, acc_sh.at[pl.ds(s * rows_per_tile + z * KK, KK)])
    if zrem:
        pltpu.sync_copy(
            xj_ab.at[pl.ds(0, zrem)],
            acc_sh.at[pl.ds(s * rows_per_tile + zfull * KK, zrem)])
    plsc.subcore_barrier()

    xj_lo = xj_ab.at[pl.ds(0, K)]
    xj_hi = xj_ab.at[pl.ds(K, K)]

    SUB = K // 2

    def _issue_xj(h, half, xj_dst, sj):
        for sub in range(2):
            pltpu.async_copy(
                x_hbm.at[sidx_v.at[h, pl.ds(half * K + sub * SUB, SUB)]],
                xj_dst.at[pl.ds(sub * SUB, SUB)], sj)

    def _issue_xi(h, half, xi_s, si):
        for sub in range(2):
            pltpu.async_copy(
                x_hbm.at[dgidx_v.at[h, pl.ds(half * K + sub * SUB, SUB)]],
                xi_s.at[pl.ds(sub * SUB, SUB)], si)

    def _issue_half(h, half, xj_dst, xi_s, sj, si):
        _issue_xj(h, half, xj_dst, sj)
        _issue_xi(h, half, xi_s, si)

    def _wait_half(h, half, xj_dst, xi_s, sj, si):
        for sub in range(2):
            pltpu.make_async_copy(
                x_hbm.at[sidx_v.at[h, pl.ds(half * K + sub * SUB, SUB)]],
                xj_dst.at[pl.ds(sub * SUB, SUB)], sj).wait()
            pltpu.make_async_copy(
                x_hbm.at[dgidx_v.at[h, pl.ds(half * K + sub * SUB, SUB)]],
                xi_s.at[pl.ds(sub * SUB, SUB)], si).wait()

    def _group(gg, _):
        base = worker_base + gg * PG
        pltpu.sync_copy(src_hbm.at[pl.ds(base, PG)], sidx_v)
        pltpu.sync_copy(dgat_hbm.at[pl.ds(base, PG)], dgidx_v)
        pltpu.sync_copy(dsca_hbm.at[pl.ds(base, PG)], dsidx_v)
        _issue_half(0, 0, xj_lo, xi_a, saj, sai)

        def _pair(h, _):
            _issue_half(h, 1, xj_hi, xi_b, sbj, sbi)
            _wait_half(h, 0, xj_lo, xi_a, saj, sai)
            _compute_chunk(xj_ab, 0, xi_a, rs_v, A_v, C_v)

            @pl.when(h + 1 < PG)
            def _():
                _issue_xi(h + 1, 0, xi_a, sai)

            _wait_half(h, 1, xj_hi, xi_b, sbj, sbi)
            _compute_chunk(xj_ab, K, xi_b, rs_v, A_v, C_v)
            pltpu.sync_copy(xj_ab, acc_sh.at[dsidx_v.at[h]], add=True)

            @pl.when(h + 1 < PG)
            def _():
                _issue_xj(h + 1, 0, xj_lo, saj)

            return 0

        lax.fori_loop(0, PG, _pair, 0)
        return 0

    lax.fori_loop(0, gpw, _group, 0)
    plsc.subcore_barrier()

    pltpu.sync_copy(acc_sh.at[pl.ds(s * rows_per_tile, rows_per_tile)],
                    out_hbm.at[c, pl.ds(s * rows_per_tile, rows_per_tile)])


def _tc_combine(x_ref, p_ref, o_ref):
    o_ref[...] = x_ref[...] * (p_ref[0] + p_ref[1])


def _pwl_tables(W1, b1, W2, b2):
    w1v = W1.reshape(-1).astype(jnp.float32)
    w2v = W2.reshape(-1).astype(jnp.float32)
    b1v = b1.astype(jnp.float32)
    nz = w1v != 0.0
    const_t = jnp.sum(jnp.where(nz, 0.0, jnp.maximum(b1v, 0.0) * w2v))
    r = jnp.where(nz, -b1v / jnp.where(nz, w1v, 1.0), jnp.inf)
    sl = jnp.where(nz, w1v * w2v, 0.0)
    ic = jnp.where(nz, b1v * w2v, 0.0)
    order = jnp.argsort(r)
    rs = r[order]
    sl, ic = sl[order], ic[order]
    pos = jnp.take(w1v, order) > 0.0
    slp = jnp.where(pos, sl, 0.0)
    icp = jnp.where(pos, ic, 0.0)
    sln = jnp.where(pos, 0.0, sl)
    icn = jnp.where(pos, 0.0, ic)
    zero1 = jnp.zeros((1,), jnp.float32)
    Ap = jnp.concatenate([zero1, jnp.cumsum(slp)])
    Cp = jnp.concatenate([zero1, jnp.cumsum(icp)])
    An = jnp.concatenate([jnp.cumsum(sln[::-1])[::-1], zero1])
    Cn = jnp.concatenate([jnp.cumsum(icn[::-1])[::-1], zero1])
    A = Ap + An
    C = Cp + Cn + const_t + b2.reshape(()).astype(jnp.float32)
    H = w1v.shape[0]
    pad = -(-(H + 1) // L) * L - (H + 1)
    A = jnp.concatenate([A, jnp.zeros((pad,), jnp.float32)])
    C = jnp.concatenate([C, jnp.zeros((pad,), jnp.float32)])
    return rs, A, C


@jax.jit
def kernel(x, edge_index, W1, b1, W2, b2):
    n, d = x.shape
    e = edge_index.shape[1]
    assert d == D and n % NS == 0

    src = edge_index[0].astype(jnp.int32)
    dst = edge_index[1].astype(jnp.int32)

    KK = 2 * K
    T = -(-e // (NS * KK * PG)) * PG
    ppw0 = max(PG, (int(T * 0.75) // PG) * PG)
    ppw1 = T - ppw0
    assert ppw1 >= PG
    e_pad = NS * T * KK
    npad = e_pad - e
    src = jnp.concatenate([src, jnp.zeros((npad,), jnp.int32)])
    dgat = jnp.concatenate([dst, jnp.zeros((npad,), jnp.int32)])
    dsca = jnp.concatenate([dst, jnp.full((npad,), n, jnp.int32)])
    src2d = src.reshape(e_pad // KK, KK)
    dgat2d = dgat.reshape(e_pad // KK, KK)
    dsca2d = dsca.reshape(e_pad // KK, KK)

    acc_rows = -(-(n + 1) // (NS * 8)) * NS * 8

    rs, A, C = _pwl_tables(W1, b1, W2, b2)

    mesh = plsc.VectorSubcoreMesh(core_axis_name="c", subcore_axis_name="s")
    partials = pl.kernel(
        functools.partial(_sc_kernel, acc_rows, ppw0, ppw1),
        out_type=jax.ShapeDtypeStruct((NC, acc_rows, D), jnp.float32),
        mesh=mesh,
        scratch_types=[
            pltpu.VMEM_SHARED((acc_rows, D), jnp.float32),
            pltpu.VMEM((KK, D), jnp.float32),
            pltpu.VMEM((K, D), jnp.float32),
            pltpu.VMEM((K, D), jnp.float32),
            pltpu.VMEM((PG, KK), jnp.int32),
            pltpu.VMEM((PG, KK), jnp.int32),
            pltpu.VMEM((PG, KK), jnp.int32),
            pltpu.VMEM(rs.shape, jnp.float32),
            pltpu.VMEM(A.shape, jnp.float32),
            pltpu.VMEM(C.shape, jnp.float32),
            pltpu.SemaphoreType.DMA,
            pltpu.SemaphoreType.DMA,
            pltpu.SemaphoreType.DMA,
            pltpu.SemaphoreType.DMA,
        ],
    )(x, src2d, dgat2d, dsca2d, rs, A, C)

    blk = 400
    out = pl.pallas_call(
        _tc_combine,
        grid=(n // blk,),
        in_specs=[
            pl.BlockSpec((blk, D), lambda i: (i, 0)),
            pl.BlockSpec((NC, blk, D), lambda i: (0, i, 0)),
        ],
        out_specs=pl.BlockSpec((blk, D), lambda i: (i, 0)),
        out_shape=jax.ShapeDtypeStruct((n, D), jnp.float32),
    )(x, partials)
    return out


# speedup vs baseline: 1.2030x; 1.0252x over previous
import functools

import jax
import jax.numpy as jnp
from jax import lax
from jax.experimental import pallas as pl
from jax.experimental.pallas import tpu as pltpu
from jax.experimental.pallas import tpu_sc as plsc

NC = 2
NS = 16
L = 16
K = 64
PG = 8
D = 128
DV = D // L


def _lanesum(v):
    for sh in (8, 4, 2, 1):
        idx = lax.rem(lax.iota(jnp.int32, L) + sh, jnp.full((L,), L, jnp.int32))
        v = v + jnp.take_along_axis(v, idx, axis=0)
    return v


def _ta(v, idx):
    return jnp.take_along_axis(v, idx, axis=0)


def _bc(v, i):
    return _ta(v, jnp.full((L,), i, jnp.int32))


def _compute_chunk(xj_b, base, xi_b, rs_v, A_v, C_v):
    lane = lax.iota(jnp.int32, L)
    rs_r = [rs_v[pl.ds(i * L, L)] for i in range(4)]
    A_r = [A_v[pl.ds(i * L, L)] for i in range(5)]
    C_r = [C_v[pl.ds(i * L, L)] for i in range(5)]
    s15, s31, s47 = _bc(rs_r[0], 15), _bc(rs_r[1], 15), _bc(rs_r[2], 15)
    for q in range(K // L):
        z = jnp.zeros((L,), jnp.float32)

        @plsc.parallel_loop(0, L, 1, unroll=4, carry=z)
        def _dot(l, se_acc):
            row = q * L + l
            rowj = base + row
            acc = xj_b[rowj, pl.ds(0, L)] * xi_b[row, pl.ds(0, L)]
            for dd in range(1, DV):
                acc = acc + (xj_b[rowj, pl.ds(dd * L, L)]
                             * xi_b[row, pl.ds(dd * L, L)])
            sp = _lanesum(acc)
            return jnp.where(lane == l, sp, se_acc)

        se = _dot * (1.0 / D)

        one = jnp.ones((L,), jnp.int32)
        zero = jnp.zeros((L,), jnp.int32)
        blk = (jnp.where(s15 < se, one, zero)
               + jnp.where(s31 < se, one, zero)
               + jnp.where(s47 < se, one, zero))

        def _probe(sel, refs, idx):
            v = jnp.where(sel >= 1, _ta(refs[1], idx), _ta(refs[0], idx))
            for i in range(2, len(refs)):
                v = jnp.where(sel >= i, _ta(refs[i], idx), v)
            return v

        kk = zero
        for b in (8, 4, 2, 1):
            v = _probe(blk, rs_r, kk + (b - 1))
            kk = jnp.where(v < se, kk + b, kk)
        kk = jnp.where(_probe(blk, rs_r, kk) < se, kk + 1, kk)
        k16 = blk * 16 + kk
        hi = jnp.right_shift(k16, 4)
        lo = jnp.bitwise_and(k16, 15)
        t = _probe(hi, A_r, lo) * se + _probe(hi, C_r, lo)
        wv = 1.0 + 1.0 / (1.0 + jnp.exp(-t))

        @plsc.parallel_loop(0, L, 1)
        def _scale(l):
            wl = jnp.take_along_axis(wv, jnp.broadcast_to(l, (L,)))
            row = base + q * L + l
            for dd in range(DV):
                xj_b[row, pl.ds(dd * L, L)] = xj_b[row, pl.ds(dd * L, L)] * wl


def _sc_kernel(acc_rows, ppw0, ppw1,
               x_hbm, src_hbm, dgat_hbm, dsca_hbm, rs_hbm, A_hbm, C_hbm,
               out_hbm,
               acc_sh, xj_ab, xi_a, xi_b, sidx_v, dgidx_v, dsidx_v,
               rs_v, A_v, C_v, saj, sai, sbj, sbi):
    c = lax.axis_index("c")
    s = lax.axis_index("s")
    KK = 2 * K
    worker_base = jnp.where(c == 0, s * ppw0, NS * ppw0 + s * ppw1)
    gpw = jnp.where(c == 0, ppw0 // PG, ppw1 // PG)

    pltpu.sync_copy(rs_hbm, rs_v)
    pltpu.sync_copy(A_hbm, A_v)
    pltpu.sync_copy(C_hbm, C_v)

    zrow = jnp.zeros((L,), jnp.float32)

    def _zero_row(i, _):
        for dd in range(DV):
            xj_ab[i, pl.ds(dd * L, L)] = zrow
        return 0

    lax.fori_loop(0, KK, _zero_row, 0)
    rows_per_tile = acc_rows // NS
    zfull, zrem = rows_per_tile // KK, rows_per_tile % KK
    for z in range(zfull):
        pltpu.sync_copy(xj_ab, acc_sh.at[pl.ds(s * rows_per_tile + z * KK, KK)])
    if zrem:
        pltpu.sync_copy(
            xj_ab.at[pl.ds(0, zrem)],
            acc_sh.at[pl.ds(s * rows_per_tile + zfull * KK, zrem)])
    plsc.subcore_barrier()

    xj_lo = xj_ab.at[pl.ds(0, K)]
    xj_hi = xj_ab.at[pl.ds(K, K)]

    SUB = K // 2

    def _issue_xj(h, half, xj_dst, sj):
        for sub in range(2):
            pltpu.async_copy(
                x_hbm.at[sidx_v.at[h, pl.ds(half * K + sub * SUB, SUB)]],
                xj_dst.at[pl.ds(sub * SUB, SUB)], sj)

    def _issue_xi(h, half, xi_s, si):
        for sub in range(2):
            pltpu.async_copy(
                x_hbm.at[dgidx_v.at[h, pl.ds(half * K + sub * SUB, SUB)]],
                xi_s.at[pl.ds(sub * SUB, SUB)], si)

    def _issue_half(h, half, xj_dst, xi_s, sj, si):
        _issue_xj(h, half, xj_dst, sj)
        _issue_xi(h, half, xi_s, si)

    def _wait_half(h, half, xj_dst, xi_s, sj, si):
        for sub in range(2):
            pltpu.make_async_copy(
                x_hbm.at[sidx_v.at[h, pl.ds(half * K + sub * SUB, SUB)]],
                xj_dst.at[pl.ds(sub * SUB, SUB)], sj).wait()
            pltpu.make_async_copy(
                x_hbm.at[dgidx_v.at[h, pl.ds(half * K + sub * SUB, SUB)]],
                xi_s.at[pl.ds(sub * SUB, SUB)], si).wait()

    def _group(gg, _):
        base = worker_base + gg * PG
        pltpu.sync_copy(src_hbm.at[pl.ds(base, PG)], sidx_v)
        pltpu.sync_copy(dgat_hbm.at[pl.ds(base, PG)], dgidx_v)
        pltpu.sync_copy(dsca_hbm.at[pl.ds(base, PG)], dsidx_v)
        _issue_half(0, 0, xj_lo, xi_a, saj, sai)

        def _pair(h, _):
            _issue_half(h, 1, xj_hi, xi_b, sbj, sbi)
            _wait_half(h, 0, xj_lo, xi_a, saj, sai)
            _compute_chunk(xj_ab, 0, xi_a, rs_v, A_v, C_v)

            @pl.when(h + 1 < PG)
            def _():
                _issue_xi(h + 1, 0, xi_a, sai)

            _wait_half(h, 1, xj_hi, xi_b, sbj, sbi)
            _compute_chunk(xj_ab, K, xi_b, rs_v, A_v, C_v)
            pltpu.sync_copy(xj_ab, acc_sh.at[dsidx_v.at[h]], add=True)

            @pl.when(h + 1 < PG)
            def _():
                _issue_xj(h + 1, 0, xj_lo, saj)

            return 0

        lax.fori_loop(0, PG, _pair, 0)
        return 0

    lax.fori_loop(0, gpw, _group, 0)
    plsc.subcore_barrier()

    pltpu.sync_copy(acc_sh.at[pl.ds(s * rows_per_tile, rows_per_tile)],
                    out_hbm.at[c, pl.ds(s * rows_per_tile, rows_per_tile)])


def _tc_combine(x_ref, p_ref, o_ref):
    o_ref[...] = x_ref[...] * (p_ref[0] + p_ref[1])


def _pwl_tables(W1, b1, W2, b2):
    w1v = W1.reshape(-1).astype(jnp.float32)
    w2v = W2.reshape(-1).astype(jnp.float32)
    b1v = b1.astype(jnp.float32)
    nz = w1v != 0.0
    const_t = jnp.sum(jnp.where(nz, 0.0, jnp.maximum(b1v, 0.0) * w2v))
    r = jnp.where(nz, -b1v / jnp.where(nz, w1v, 1.0), jnp.inf)
    sl = jnp.where(nz, w1v * w2v, 0.0)
    ic = jnp.where(nz, b1v * w2v, 0.0)
    order = jnp.argsort(r)
    rs = r[order]
    sl, ic = sl[order], ic[order]
    pos = jnp.take(w1v, order) > 0.0
    slp = jnp.where(pos, sl, 0.0)
    icp = jnp.where(pos, ic, 0.0)
    sln = jnp.where(pos, 0.0, sl)
    icn = jnp.where(pos, 0.0, ic)
    zero1 = jnp.zeros((1,), jnp.float32)
    Ap = jnp.concatenate([zero1, jnp.cumsum(slp)])
    Cp = jnp.concatenate([zero1, jnp.cumsum(icp)])
    An = jnp.concatenate([jnp.cumsum(sln[::-1])[::-1], zero1])
    Cn = jnp.concatenate([jnp.cumsum(icn[::-1])[::-1], zero1])
    A = Ap + An
    C = Cp + Cn + const_t + b2.reshape(()).astype(jnp.float32)
    H = w1v.shape[0]
    pad = -(-(H + 1) // L) * L - (H + 1)
    A = jnp.concatenate([A, jnp.zeros((pad,), jnp.float32)])
    C = jnp.concatenate([C, jnp.zeros((pad,), jnp.float32)])
    return rs, A, C


@jax.jit
def kernel(x, edge_index, W1, b1, W2, b2):
    n, d = x.shape
    e = edge_index.shape[1]
    assert d == D and n % NS == 0

    src = edge_index[0].astype(jnp.int32)
    dst = edge_index[1].astype(jnp.int32)

    KK = 2 * K
    T = -(-e // (NS * KK * PG)) * PG
    ppw0 = max(PG, (int(T * 0.80) // PG) * PG)
    ppw1 = T - ppw0
    assert ppw1 >= PG
    e_pad = NS * T * KK
    npad = e_pad - e
    src = jnp.concatenate([src, jnp.zeros((npad,), jnp.int32)])
    dgat = jnp.concatenate([dst, jnp.zeros((npad,), jnp.int32)])
    dsca = jnp.concatenate([dst, jnp.full((npad,), n, jnp.int32)])
    src2d = src.reshape(e_pad // KK, KK)
    dgat2d = dgat.reshape(e_pad // KK, KK)
    dsca2d = dsca.reshape(e_pad // KK, KK)

    acc_rows = -(-(n + 1) // (NS * 8)) * NS * 8

    rs, A, C = _pwl_tables(W1, b1, W2, b2)

    mesh = plsc.VectorSubcoreMesh(core_axis_name="c", subcore_axis_name="s")
    partials = pl.kernel(
        functools.partial(_sc_kernel, acc_rows, ppw0, ppw1),
        out_type=jax.ShapeDtypeStruct((NC, acc_rows, D), jnp.float32),
        mesh=mesh,
        scratch_types=[
            pltpu.VMEM_SHARED((acc_rows, D), jnp.float32),
            pltpu.VMEM((KK, D), jnp.float32),
            pltpu.VMEM((K, D), jnp.float32),
            pltpu.VMEM((K, D), jnp.float32),
            pltpu.VMEM((PG, KK), jnp.int32),
            pltpu.VMEM((PG, KK), jnp.int32),
            pltpu.VMEM((PG, KK), jnp.int32),
            pltpu.VMEM(rs.shape, jnp.float32),
            pltpu.VMEM(A.shape, jnp.float32),
            pltpu.VMEM(C.shape, jnp.float32),
            pltpu.SemaphoreType.DMA,
            pltpu.SemaphoreType.DMA,
            pltpu.SemaphoreType.DMA,
            pltpu.SemaphoreType.DMA,
        ],
    )(x, src2d, dgat2d, dsca2d, rs, A, C)

    blk = 400
    out = pl.pallas_call(
        _tc_combine,
        grid=(n // blk,),
        in_specs=[
            pl.BlockSpec((blk, D), lambda i: (i, 0)),
            pl.BlockSpec((NC, blk, D), lambda i: (0, i, 0)),
        ],
        out_specs=pl.BlockSpec((blk, D), lambda i: (i, 0)),
        out_shape=jax.ShapeDtypeStruct((n, D), jnp.float32),
    )(x, partials)
    return out
